# baseline (speedup 1.0000x reference)
import functools

import jax
import jax.numpy as jnp
from jax import lax
from jax.experimental import pallas as pl
from jax.experimental.pallas import tpu as pltpu
from jax.experimental.pallas import tpu_sc as plsc

N_CORES = 2
N_TILES = 16


def _chunk_size(per_tile, max_b):
    for b in range(max_b, 3, -4):
        if per_tile % b == 0 and (per_tile // b) % 2 == 0:
            return b
    return None


@functools.lru_cache(maxsize=None)
def _make_sc_aggr(n_in, n, d2, e, with_deg):
    per_tile = e // N_TILES
    B = _chunk_size(per_tile, 100)
    n_chunks = per_tile // B
    rt = n // N_TILES

    mesh = plsc.VectorSubcoreMesh(
        core_axis_name="c", subcore_axis_name="s",
        num_cores=N_CORES, num_subcores=N_TILES)

    out_type = [jax.ShapeDtypeStruct((n, d2), jnp.float32),
                jax.ShapeDtypeStruct((n, d2), jnp.float32)]
    scratch = [
        pltpu.VMEM((2, 2, B), jnp.int32),
        pltpu.VMEM((2, B, d2), jnp.float32),
        pltpu.VMEM_SHARED((n, d2), jnp.float32),
        pltpu.SemaphoreType.DMA,
        pltpu.SemaphoreType.DMA,
        pltpu.SemaphoreType.DMA,
        pltpu.SemaphoreType.DMA,
    ]
    if with_deg:
        out_type.append(jax.ShapeDtypeStruct((n, 16), jnp.float32))
        out_type.append(jax.ShapeDtypeStruct((n, 16), jnp.float32))
        scratch += [
            pltpu.VMEM((B, 16), jnp.float32),
            pltpu.VMEM_SHARED((n, 16), jnp.float32),
        ]

    def body(h0, h1, edges4, z_acc, z_deg, ones_h,
             sum0, sum1, deg_out0, deg_out1, idx_v, rows_v, acc_sh,
             gsem0, gsem1, isem0, isem1,
             ones_v=None, deg_sh=None):
        c = lax.axis_index("c")
        s = lax.axis_index("s")
        sl = pl.ds(s * rt, rt)

        pltpu.sync_copy(z_acc, acc_sh.at[sl])

        if with_deg:
            pltpu.sync_copy(z_deg, deg_sh.at[sl])
            pltpu.sync_copy(ones_h, ones_v)

        plsc.subcore_barrier()

        def run(h_ref, deg_even, deg_odd):
            i0, i1 = idx_v.at[0], idx_v.at[1]
            r0, r1 = rows_v.at[0], rows_v.at[1]

            def scat(i_ref, r_ref, do_deg):
                pltpu.sync_copy(r_ref, acc_sh.at[i_ref.at[1]], add=True)
                if do_deg:
                    pltpu.sync_copy(ones_v, deg_sh.at[i_ref.at[1]], add=True)

            pltpu.sync_copy(edges4.at[s, 0], i0)
            pltpu.async_copy(h_ref.at[i0.at[0]], r0, gsem0)
            pltpu.async_copy(edges4.at[s, 1], i1, isem1)

            def pair(t, carry):
                j = 2 * t
                pltpu.make_async_copy(edges4.at[s, 0], i1, isem1).wait()
                pltpu.make_async_copy(h_ref.at[i0.at[0]], r0, gsem0).wait()
                pltpu.async_copy(h_ref.at[i1.at[0]], r1, gsem1)
                scat(i0, r0, deg_even)

                @pl.when(j + 2 < n_chunks)
                def _():
                    pltpu.async_copy(edges4.at[s, j + 2], i0, isem0)

                @pl.when(j + 2 < n_chunks)
                def _():
                    pltpu.make_async_copy(edges4.at[s, 0], i0, isem0).wait()
                    pltpu.async_copy(h_ref.at[i0.at[0]], r0, gsem0)

                pltpu.make_async_copy(h_ref.at[i1.at[0]], r1, gsem1).wait()
                scat(i1, r1, deg_odd)

                @pl.when(j + 3 < n_chunks)
                def _():
                    pltpu.async_copy(edges4.at[s, j + 3], i1, isem1)
                return carry

            lax.fori_loop(0, n_chunks // 2, pair, 0)

        @pl.when(c == 0)
        def _():
            run(h0, with_deg, False)

        @pl.when(c == 1)
        def _():
            run(h1, False, with_deg)

        plsc.subcore_barrier()

        @pl.when(c == 0)
        def _():
            pltpu.sync_copy(acc_sh.at[sl], sum0.at[sl])
            if with_deg:
                pltpu.sync_copy(deg_sh.at[sl], deg_out0.at[sl])

        @pl.when(c == 1)
        def _():
            pltpu.sync_copy(acc_sh.at[sl], sum1.at[sl])
            if with_deg:
                pltpu.sync_copy(deg_sh.at[sl], deg_out1.at[sl])

    if with_deg:
        def body_wd(h0, h1, edges4, z_acc, z_deg, ones_h,
                    sum0, sum1, deg_out0, deg_out1,
                    idx_v, rows_v, acc_sh, gsem0, gsem1, isem0, isem1,
                    ones_v, deg_sh):
            body(h0, h1, edges4, z_acc, z_deg, ones_h,
                 sum0, sum1, deg_out0, deg_out1, idx_v, rows_v, acc_sh,
                 gsem0, gsem1, isem0, isem1, ones_v=ones_v, deg_sh=deg_sh)
        fn = pl.kernel(body_wd, out_type=tuple(out_type), mesh=mesh,
                       scratch_types=tuple(scratch),
                       compiler_params=pltpu.CompilerParams(
                           use_tc_tiling_on_sc=False))
    else:
        def body_nd(h0, h1, edges4, z_acc, z_deg, ones_h, sum0, sum1,
                    idx_v, rows_v, acc_sh, gsem0, gsem1, isem0, isem1):
            body(h0, h1, edges4, z_acc, z_deg, ones_h,
                 sum0, sum1, None, None, idx_v, rows_v, acc_sh,
                 gsem0, gsem1, isem0, isem1)
        fn = pl.kernel(body_nd, out_type=tuple(out_type), mesh=mesh,
                       scratch_types=tuple(scratch),
                       compiler_params=pltpu.CompilerParams(
                           use_tc_tiling_on_sc=False))
    return fn, B


def _sc_aggr(h0, h1, src, dst, n_out, z_acc, z_deg, ones_h, with_deg):
    n_in, d2 = h0.shape
    e = src.shape[0]
    fn, B = _make_sc_aggr(n_in, n_out, d2, e, with_deg)
    per_tile = e // N_TILES
    n_chunks = per_tile // B
    edges4 = jnp.stack([src.reshape(N_TILES, n_chunks, B),
                        dst.reshape(N_TILES, n_chunks, B)], axis=2)
    return fn(h0, h1, edges4, z_acc, z_deg, ones_h)


def _bdot(a, w):
    return lax.dot_general(a, w, (((1,), (1,)), ((), ())),
                           preferred_element_type=jnp.float32)


def _tc_layer_body(h0, h1, s0, s1, deg0, deg1, w1, b1, w2, b2, out0, out1):
    inv = 1.0 / jnp.maximum(deg0[:, 0:1] + deg1[:, 0:1], 1.0)
    hprev = jnp.concatenate([h0[...], h1[...]], axis=1)
    aggr = jnp.concatenate([s0[...], s1[...]], axis=1) * inv
    cat = jnp.concatenate([hprev, aggr], axis=1)
    hid = jnp.maximum(_bdot(cat, w1[...]) + b1[...], 0.0)
    out = jnp.maximum(_bdot(hid, w2[...]) + b2[...], 0.0)
    d2 = out.shape[1] // 2
    out0[...] = out[:, :d2]
    out1[...] = out[:, d2:]


@functools.lru_cache(maxsize=None)
def _make_tc_layer(n, d, h_dim, rows):
    d2 = d // 2
    grid = (n // rows,)
    full = lambda shape: pl.BlockSpec(shape, lambda i: (0, 0))
    row_blk = lambda cols: pl.BlockSpec((rows, cols), lambda i: (i, 0))
    return pl.pallas_call(
        _tc_layer_body,
        grid=grid,
        in_specs=[row_blk(d2), row_blk(d2), row_blk(d2), row_blk(d2),
                  row_blk(16), row_blk(16),
                  full((h_dim, 2 * d)), full((1, h_dim)),
                  full((d, h_dim)), full((1, d))],
        out_specs=[row_blk(d2), row_blk(d2)],
        out_shape=[jax.ShapeDtypeStruct((n, d2), jnp.float32),
                   jax.ShapeDtypeStruct((n, d2), jnp.float32)],
    )


def _layer2_pool_body(g_num, rows, h0, h1, s0, s1, deg0, deg1, w1, b1,
                      w2, b2, batch_blk, batch, wh1, bh1, wh2, bh2,
                      out_ref, h2_acc, sum_acc, cnt_acc):
    i = pl.program_id(0)

    @pl.when(i == 0)
    def _():
        sum_acc[...] = jnp.zeros_like(sum_acc)
        cnt_acc[...] = jnp.zeros_like(cnt_acc)

    inv = 1.0 / jnp.maximum(deg0[:, 0:1] + deg1[:, 0:1], 1.0)
    hprev = jnp.concatenate([h0[...], h1[...]], axis=1)
    aggr = jnp.concatenate([s0[...], s1[...]], axis=1) * inv
    cat = jnp.concatenate([hprev, aggr], axis=1)
    hid = jnp.maximum(_bdot(cat, w1[...]) + b1[...], 0.0)
    out = jnp.maximum(_bdot(hid, w2[...]) + b2[...], 0.0)
    h2_acc[pl.ds(i * rows, rows), :] = out

    b = batch_blk[...].reshape(1, rows)
    gid = lax.broadcasted_iota(jnp.int32, (g_num, rows), 0)
    m = jnp.where(gid == jnp.broadcast_to(b, (g_num, rows)), 1.0, 0.0)
    sum_acc[...] += jnp.dot(m, out, preferred_element_type=jnp.float32)
    cnt_acc[...] += jnp.broadcast_to(
        jnp.sum(m, axis=1, keepdims=True), cnt_acc.shape)

    @pl.when(i == pl.num_programs(0) - 1)
    def _():
        n = h2_acc.shape[0]
        bf = batch[...]
        gidf = lax.broadcasted_iota(jnp.int32, (g_num, n), 0)
        mf = jnp.where(gidf == jnp.broadcast_to(bf, (g_num, n)), 1.0, 0.0)
        cnt = jnp.maximum(cnt_acc[:, 0:1], 1.0)
        mean = sum_acc[...] / cnt
        mean_pn = lax.dot_general(mf, mean, (((0,), (0,)), ((), ())),
                                  preferred_element_type=jnp.float32)
        diff = h2_acc[...] - mean_pn
        var = jnp.dot(mf, diff * diff,
                      preferred_element_type=jnp.float32) / cnt
        std = jnp.sqrt(jnp.clip(var, 1e-12))
        g = jnp.concatenate([mean, std], axis=1)
        hid2 = jnp.maximum(_bdot(g, wh1[...]) + bh1[...], 0.0)
        out_ref[...] = _bdot(hid2, wh2[...]) + bh2[...]


@functools.lru_cache(maxsize=None)
def _make_layer2_pool(n, d, h_dim, g_num, rows, out_pad):
    d2 = d // 2
    grid = (n // rows,)
    full = lambda shape: pl.BlockSpec(shape, lambda i: (0, 0))
    row_blk = lambda cols: pl.BlockSpec((rows, cols), lambda i: (i, 0))
    return pl.pallas_call(
        functools.partial(_layer2_pool_body, g_num, rows),
        grid=grid,
        in_specs=[row_blk(d2), row_blk(d2), row_blk(d2), row_blk(d2),
                  row_blk(16), row_blk(16),
                  full((h_dim, 2 * d)), full((1, h_dim)),
                  full((d, h_dim)), full((1, d)),
                  pl.BlockSpec((1, 1, rows), lambda i: (i, 0, 0)),
                  full((1, n)),
                  full((h_dim, 2 * d)), full((1, h_dim)),
                  full((out_pad, h_dim)), full((1, out_pad))],
        out_specs=pl.BlockSpec((g_num, out_pad), lambda i: (0, 0)),
        out_shape=jax.ShapeDtypeStruct((g_num, out_pad), jnp.float32),
        scratch_shapes=[pltpu.VMEM((n, d), jnp.float32),
                        pltpu.VMEM((g_num, d), jnp.float32),
                        pltpu.VMEM((g_num, 128), jnp.float32)],
    )


def _pool_head_body(g_num, h20, h21, batch, wh1t, bh1, wh2t, bh2,
                    out_ref):
    h2 = jnp.concatenate([h20[...], h21[...]], axis=1)
    n = h2.shape[0]
    b = batch[...]
    gid = lax.broadcasted_iota(jnp.int32, (g_num, n), 0)
    m = jnp.where(gid == jnp.broadcast_to(b, (g_num, n)), 1.0, 0.0)
    cnt = jnp.sum(m, axis=1, keepdims=True)
    inv = 1.0 / jnp.maximum(cnt, 1.0)
    mean = jnp.dot(m, h2, preferred_element_type=jnp.float32) * inv
    mean_pn = lax.dot_general(m, mean, (((0,), (0,)), ((), ())),
                              preferred_element_type=jnp.float32)
    diff = h2 - mean_pn
    var = jnp.dot(m, diff * diff, preferred_element_type=jnp.float32) * inv
    std = jnp.sqrt(jnp.clip(var, 1e-12))
    g = jnp.concatenate([mean, std], axis=1)
    hid = jnp.maximum(
        jnp.dot(g, wh1t[...], preferred_element_type=jnp.float32) + bh1[...],
        0.0)
    out_ref[...] = (jnp.dot(hid, wh2t[...], preferred_element_type=jnp.float32)
                    + bh2[...])


@functools.lru_cache(maxsize=None)
def _make_pool_head(n, d, h_dim, g_num, out_pad):
    return pl.pallas_call(
        functools.partial(_pool_head_body, g_num),
        out_shape=jax.ShapeDtypeStruct((g_num, out_pad), jnp.float32),
    )


def kernel(x, edge_index, batch, W1a, b1a, W2a, b2a, W1b, b1b, W2b, b2b,
           Wh1, bh1, Wh2, bh2):
    n, d = x.shape
    e = edge_index.shape[1]
    h_dim = W1a.shape[0]
    g_num = 64
    d2 = d // 2
    out_pad = 128

    np_ = ((n + 127) // 128) * 128
    rt = np_ // N_TILES

    src, dst = edge_index[0], edge_index[1]
    z_acc = jnp.zeros((rt, d2), jnp.float32)
    z_deg = jnp.zeros((rt, 16), jnp.float32)
    ones_h = jnp.ones((_chunk_size(e // N_TILES, 100), 16), jnp.float32)

    x0, x1 = x[:, :d2], x[:, d2:]
    s0, s1, dg0, dg1 = _sc_aggr(x0, x1, src, dst, np_, z_acc, z_deg,
                                ones_h, True)
    rows = next(r for r in range(2048, 7, -8) if n % r == 0)
    layer = _make_tc_layer(n, d, h_dim, rows)
    h1 = layer(x0, x1, s0, s1, dg0, dg1,
               W1a, b1a.reshape(1, -1), W2a, b2a.reshape(1, -1))

    t0, t1 = _sc_aggr(h1[0], h1[1], src, dst, np_, z_acc, z_deg, ones_h,
                      False)
    wh2p = jnp.pad(Wh2, ((0, out_pad - Wh2.shape[0]), (0, 0)))
    bh2p = jnp.pad(bh2, (0, out_pad - bh2.shape[0]))
    l2p = _make_layer2_pool(n, d, h_dim, g_num, rows, out_pad)
    out = l2p(h1[0], h1[1], t0, t1, dg0, dg1,
              W1b, b1b.reshape(1, -1), W2b, b2b.reshape(1, -1),
              batch.reshape(n // rows, 1, rows), batch.reshape(1, -1),
              Wh1, bh1.reshape(1, -1), wh2p, bh2p.reshape(1, -1))
    return out[:, 0]


# speedup vs baseline: 1.0059x; 1.0059x over previous
import functools

import jax
import jax.numpy as jnp
from jax import lax
from jax.experimental import pallas as pl
from jax.experimental.pallas import tpu as pltpu
from jax.experimental.pallas import tpu_sc as plsc

N_CORES = 2
N_TILES = 16


def _chunk_size(per_tile, max_b):
    for b in range(max_b, 3, -4):
        if per_tile % b == 0 and (per_tile // b) % 2 == 0:
            return b
    return None


@functools.lru_cache(maxsize=None)
def _make_sc_aggr(n_in, n, d2, e, with_deg):
    per_tile = e // N_TILES
    B = _chunk_size(per_tile, 100)
    n_chunks = per_tile // B
    rt = n // N_TILES

    mesh = plsc.VectorSubcoreMesh(
        core_axis_name="c", subcore_axis_name="s",
        num_cores=N_CORES, num_subcores=N_TILES)

    out_type = [jax.ShapeDtypeStruct((n, d2), jnp.float32),
                jax.ShapeDtypeStruct((n, d2), jnp.float32)]
    scratch = [
        pltpu.VMEM((2, 2, B), jnp.int32),
        pltpu.VMEM((2, B, d2), jnp.float32),
        pltpu.VMEM_SHARED((n, d2), jnp.float32),
        pltpu.SemaphoreType.DMA,
        pltpu.SemaphoreType.DMA,
        pltpu.SemaphoreType.DMA,
        pltpu.SemaphoreType.DMA,
    ]
    if with_deg:
        out_type.append(jax.ShapeDtypeStruct((n, 16), jnp.float32))
        scratch += [
            pltpu.VMEM((B, 16), jnp.float32),
            pltpu.VMEM_SHARED((n, 16), jnp.float32),
        ]

    def body(h0, h1, edges4, z_acc, z_deg, ones_h,
             sum0, sum1, deg_out, idx_v, rows_v, acc_sh,
             gsem0, gsem1, isem0, isem1,
             ones_v=None, deg_sh=None):
        c = lax.axis_index("c")
        s = lax.axis_index("s")
        sl = pl.ds(s * rt, rt)

        pltpu.sync_copy(z_acc, acc_sh.at[sl])

        if with_deg:
            @pl.when(c == 0)
            def _():
                pltpu.sync_copy(z_deg, deg_sh.at[sl])
                pltpu.sync_copy(ones_h, ones_v)

        plsc.subcore_barrier()

        def run(h_ref, do_deg):
            i0, i1 = idx_v.at[0], idx_v.at[1]
            r0, r1 = rows_v.at[0], rows_v.at[1]

            def scat(i_ref, r_ref):
                pltpu.sync_copy(r_ref, acc_sh.at[i_ref.at[1]], add=True)
                if do_deg:
                    pltpu.sync_copy(ones_v, deg_sh.at[i_ref.at[1]], add=True)

            pltpu.sync_copy(edges4.at[s, 0], i0)
            pltpu.async_copy(h_ref.at[i0.at[0]], r0, gsem0)
            pltpu.async_copy(edges4.at[s, 1], i1, isem1)

            def pair(t, carry):
                j = 2 * t
                pltpu.make_async_copy(edges4.at[s, 0], i1, isem1).wait()
                pltpu.make_async_copy(h_ref.at[i0.at[0]], r0, gsem0).wait()
                pltpu.async_copy(h_ref.at[i1.at[0]], r1, gsem1)
                scat(i0, r0)

                @pl.when(j + 2 < n_chunks)
                def _():
                    pltpu.async_copy(edges4.at[s, j + 2], i0, isem0)

                @pl.when(j + 2 < n_chunks)
                def _():
                    pltpu.make_async_copy(edges4.at[s, 0], i0, isem0).wait()
                    pltpu.async_copy(h_ref.at[i0.at[0]], r0, gsem0)

                pltpu.make_async_copy(h_ref.at[i1.at[0]], r1, gsem1).wait()
                scat(i1, r1)

                @pl.when(j + 3 < n_chunks)
                def _():
                    pltpu.async_copy(edges4.at[s, j + 3], i1, isem1)
                return carry

            lax.fori_loop(0, n_chunks // 2, pair, 0)

        @pl.when(c == 0)
        def _():
            run(h0, with_deg)

        @pl.when(c == 1)
        def _():
            run(h1, False)

        plsc.subcore_barrier()

        @pl.when(c == 0)
        def _():
            pltpu.sync_copy(acc_sh.at[sl], sum0.at[sl])
            if with_deg:
                pltpu.sync_copy(deg_sh.at[sl], deg_out.at[sl])

        @pl.when(c == 1)
        def _():
            pltpu.sync_copy(acc_sh.at[sl], sum1.at[sl])

    if with_deg:
        def body_wd(h0, h1, edges4, z_acc, z_deg, ones_h,
                    sum0, sum1, deg_out,
                    idx_v, rows_v, acc_sh, gsem0, gsem1, isem0, isem1,
                    ones_v, deg_sh):
            body(h0, h1, edges4, z_acc, z_deg, ones_h,
                 sum0, sum1, deg_out, idx_v, rows_v, acc_sh,
                 gsem0, gsem1, isem0, isem1, ones_v=ones_v, deg_sh=deg_sh)
        fn = pl.kernel(body_wd, out_type=tuple(out_type), mesh=mesh,
                       scratch_types=tuple(scratch),
                       compiler_params=pltpu.CompilerParams(
                           use_tc_tiling_on_sc=False))
    else:
        def body_nd(h0, h1, edges4, z_acc, z_deg, ones_h, sum0, sum1,
                    idx_v, rows_v, acc_sh, gsem0, gsem1, isem0, isem1):
            body(h0, h1, edges4, z_acc, z_deg, ones_h,
                 sum0, sum1, None, idx_v, rows_v, acc_sh,
                 gsem0, gsem1, isem0, isem1)
        fn = pl.kernel(body_nd, out_type=tuple(out_type), mesh=mesh,
                       scratch_types=tuple(scratch),
                       compiler_params=pltpu.CompilerParams(
                           use_tc_tiling_on_sc=False))
    return fn, B


def _sc_aggr(h0, h1, src, dst, n_out, z_acc, z_deg, ones_h, with_deg):
    n_in, d2 = h0.shape
    e = src.shape[0]
    fn, B = _make_sc_aggr(n_in, n_out, d2, e, with_deg)
    per_tile = e // N_TILES
    n_chunks = per_tile // B
    edges4 = jnp.stack([src.reshape(N_TILES, n_chunks, B),
                        dst.reshape(N_TILES, n_chunks, B)], axis=2)
    return fn(h0, h1, edges4, z_acc, z_deg, ones_h)


def _bdot(a, w):
    return lax.dot_general(a, w, (((1,), (1,)), ((), ())),
                           preferred_element_type=jnp.float32)


def _tc_layer_body(h0, h1, s0, s1, deg, w1, b1, w2, b2, out0, out1):
    inv = 1.0 / jnp.maximum(deg[:, 0:1], 1.0)
    hprev = jnp.concatenate([h0[...], h1[...]], axis=1)
    aggr = jnp.concatenate([s0[...], s1[...]], axis=1) * inv
    cat = jnp.concatenate([hprev, aggr], axis=1)
    hid = jnp.maximum(_bdot(cat, w1[...]) + b1[...], 0.0)
    out = jnp.maximum(_bdot(hid, w2[...]) + b2[...], 0.0)
    d2 = out.shape[1] // 2
    out0[...] = out[:, :d2]
    out1[...] = out[:, d2:]


@functools.lru_cache(maxsize=None)
def _make_tc_layer(n, d, h_dim, rows):
    d2 = d // 2
    grid = (n // rows,)
    full = lambda shape: pl.BlockSpec(shape, lambda i: (0, 0))
    row_blk = lambda cols: pl.BlockSpec((rows, cols), lambda i: (i, 0))
    return pl.pallas_call(
        _tc_layer_body,
        grid=grid,
        in_specs=[row_blk(d2), row_blk(d2), row_blk(d2), row_blk(d2),
                  row_blk(16),
                  full((h_dim, 2 * d)), full((1, h_dim)),
                  full((d, h_dim)), full((1, d))],
        out_specs=[row_blk(d2), row_blk(d2)],
        out_shape=[jax.ShapeDtypeStruct((n, d2), jnp.float32),
                   jax.ShapeDtypeStruct((n, d2), jnp.float32)],
    )


def _layer2_pool_body(g_num, rows, h0, h1, s0, s1, deg, w1, b1,
                      w2, b2, batch_blk, batch, wh1, bh1, wh2, bh2,
                      out_ref, h2_acc, sum_acc, cnt_acc):
    i = pl.program_id(0)

    @pl.when(i == 0)
    def _():
        sum_acc[...] = jnp.zeros_like(sum_acc)
        cnt_acc[...] = jnp.zeros_like(cnt_acc)

    inv = 1.0 / jnp.maximum(deg[:, 0:1], 1.0)
    hprev = jnp.concatenate([h0[...], h1[...]], axis=1)
    aggr = jnp.concatenate([s0[...], s1[...]], axis=1) * inv
    cat = jnp.concatenate([hprev, aggr], axis=1)
    hid = jnp.maximum(_bdot(cat, w1[...]) + b1[...], 0.0)
    out = jnp.maximum(_bdot(hid, w2[...]) + b2[...], 0.0)
    h2_acc[pl.ds(i * rows, rows), :] = out

    b = batch_blk[...].reshape(1, rows)
    gid = lax.broadcasted_iota(jnp.int32, (g_num, rows), 0)
    m = jnp.where(gid == jnp.broadcast_to(b, (g_num, rows)), 1.0, 0.0)
    sum_acc[...] += jnp.dot(m, out, preferred_element_type=jnp.float32)
    cnt_acc[...] += jnp.broadcast_to(
        jnp.sum(m, axis=1, keepdims=True), cnt_acc.shape)

    @pl.when(i == pl.num_programs(0) - 1)
    def _():
        n = h2_acc.shape[0]
        bf = batch[...]
        gidf = lax.broadcasted_iota(jnp.int32, (g_num, n), 0)
        mf = jnp.where(gidf == jnp.broadcast_to(bf, (g_num, n)), 1.0, 0.0)
        cnt = jnp.maximum(cnt_acc[:, 0:1], 1.0)
        mean = sum_acc[...] / cnt
        mean_pn = lax.dot_general(mf, mean, (((0,), (0,)), ((), ())),
                                  preferred_element_type=jnp.float32)
        diff = h2_acc[...] - mean_pn
        var = jnp.dot(mf, diff * diff,
                      preferred_element_type=jnp.float32) / cnt
        std = jnp.sqrt(jnp.clip(var, 1e-12))
        g = jnp.concatenate([mean, std], axis=1)
        hid2 = jnp.maximum(_bdot(g, wh1[...]) + bh1[...], 0.0)
        out_ref[...] = _bdot(hid2, wh2[...]) + bh2[...]


@functools.lru_cache(maxsize=None)
def _make_layer2_pool(n, d, h_dim, g_num, rows, out_pad):
    d2 = d // 2
    grid = (n // rows,)
    full = lambda shape: pl.BlockSpec(shape, lambda i: (0, 0))
    row_blk = lambda cols: pl.BlockSpec((rows, cols), lambda i: (i, 0))
    return pl.pallas_call(
        functools.partial(_layer2_pool_body, g_num, rows),
        grid=grid,
        in_specs=[row_blk(d2), row_blk(d2), row_blk(d2), row_blk(d2),
                  row_blk(16),
                  full((h_dim, 2 * d)), full((1, h_dim)),
                  full((d, h_dim)), full((1, d)),
                  pl.BlockSpec((1, 1, rows), lambda i: (i, 0, 0)),
                  full((1, n)),
                  full((h_dim, 2 * d)), full((1, h_dim)),
                  full((out_pad, h_dim)), full((1, out_pad))],
        out_specs=pl.BlockSpec((g_num, out_pad), lambda i: (0, 0)),
        out_shape=jax.ShapeDtypeStruct((g_num, out_pad), jnp.float32),
        scratch_shapes=[pltpu.VMEM((n, d), jnp.float32),
                        pltpu.VMEM((g_num, d), jnp.float32),
                        pltpu.VMEM((g_num, 128), jnp.float32)],
    )


def _pool_head_body(g_num, h20, h21, batch, wh1t, bh1, wh2t, bh2,
                    out_ref):
    h2 = jnp.concatenate([h20[...], h21[...]], axis=1)
    n = h2.shape[0]
    b = batch[...]
    gid = lax.broadcasted_iota(jnp.int32, (g_num, n), 0)
    m = jnp.where(gid == jnp.broadcast_to(b, (g_num, n)), 1.0, 0.0)
    cnt = jnp.sum(m, axis=1, keepdims=True)
    inv = 1.0 / jnp.maximum(cnt, 1.0)
    mean = jnp.dot(m, h2, preferred_element_type=jnp.float32) * inv
    mean_pn = lax.dot_general(m, mean, (((0,), (0,)), ((), ())),
                              preferred_element_type=jnp.float32)
    diff = h2 - mean_pn
    var = jnp.dot(m, diff * diff, preferred_element_type=jnp.float32) * inv
    std = jnp.sqrt(jnp.clip(var, 1e-12))
    g = jnp.concatenate([mean, std], axis=1)
    hid = jnp.maximum(
        jnp.dot(g, wh1t[...], preferred_element_type=jnp.float32) + bh1[...],
        0.0)
    out_ref[...] = (jnp.dot(hid, wh2t[...], preferred_element_type=jnp.float32)
                    + bh2[...])


@functools.lru_cache(maxsize=None)
def _make_pool_head(n, d, h_dim, g_num, out_pad):
    return pl.pallas_call(
        functools.partial(_pool_head_body, g_num),
        out_shape=jax.ShapeDtypeStruct((g_num, out_pad), jnp.float32),
    )


def kernel(x, edge_index, batch, W1a, b1a, W2a, b2a, W1b, b1b, W2b, b2b,
           Wh1, bh1, Wh2, bh2):
    n, d = x.shape
    e = edge_index.shape[1]
    h_dim = W1a.shape[0]
    g_num = 64
    d2 = d // 2
    out_pad = 128

    np_ = ((n + 127) // 128) * 128
    rt = np_ // N_TILES

    src, dst = edge_index[0], edge_index[1]
    z_acc = jnp.zeros((rt, d2), jnp.float32)
    z_deg = jnp.zeros((rt, 16), jnp.float32)
    ones_h = jnp.ones((_chunk_size(e // N_TILES, 100), 16), jnp.float32)

    x0, x1 = x[:, :d2], x[:, d2:]
    s0, s1, deg16 = _sc_aggr(x0, x1, src, dst, np_, z_acc, z_deg,
                             ones_h, True)
    rows = next(r for r in range(2048, 7, -8) if n % r == 0)
    layer = _make_tc_layer(n, d, h_dim, rows)
    h1 = layer(x0, x1, s0, s1, deg16,
               W1a, b1a.reshape(1, -1), W2a, b2a.reshape(1, -1))

    t0, t1 = _sc_aggr(h1[0], h1[1], src, dst, np_, z_acc, z_deg, ones_h,
                      False)
    wh2p = jnp.pad(Wh2, ((0, out_pad - Wh2.shape[0]), (0, 0)))
    bh2p = jnp.pad(bh2, (0, out_pad - bh2.shape[0]))
    l2p = _make_layer2_pool(n, d, h_dim, g_num, rows, out_pad)
    out = l2p(h1[0], h1[1], t0, t1, deg16,
              W1b, b1b.reshape(1, -1), W2b, b2b.reshape(1, -1),
              batch.reshape(n // rows, 1, rows), batch.reshape(1, -1),
              Wh1, bh1.reshape(1, -1), wh2p, bh2p.reshape(1, -1))
    return out[:, 0]
